# baseline (speedup 1.0000x reference)
import functools

import jax
import jax.numpy as jnp
from jax import lax
from jax.experimental import pallas as pl
from jax.experimental.pallas import tpu as pltpu
from jax.experimental.pallas import tpu_sc as plsc

_NC, _NS, _L = 2, 16, 16
_NW = _NC * _NS
_CHUNK = 32


def _sc_body(tid_hbm, emb_hbm, tok_hbm, out_hbm, idx_v, row_v, buf, sem):
    wid = lax.axis_index("s") * _NC + lax.axis_index("c")
    rows, d_model = tok_hbm.shape
    rows_per_w = rows // _NW
    base = wid * rows_per_w

    pltpu.sync_copy(tid_hbm, idx_v)
    pltpu.async_copy(emb_hbm.at[idx_v], row_v, sem).wait()

    @pl.loop(0, rows_per_w // _CHUNK)
    def _chunk(c):
        r0 = base + c * _CHUNK
        pltpu.sync_copy(tok_hbm.at[pl.ds(r0, _CHUNK)], buf)

        @pl.loop(0, d_model // _L)
        def _d(d):
            col = d * _L
            rv = row_v[0, pl.ds(col, _L)]

            @plsc.parallel_loop(0, _CHUNK, unroll=8)
            def _r(r):
                buf[r, pl.ds(col, _L)] += rv

        pltpu.sync_copy(buf, out_hbm.at[pl.ds(r0, _CHUNK)])


def kernel(tokens, embed_weight, type_id):
    B, N, D = tokens.shape
    rows = B * N
    flat = tokens.reshape(rows, D)
    tid_vec = jnp.full((8,), type_id, jnp.int32)
    mesh = plsc.VectorSubcoreMesh(
        core_axis_name="c", subcore_axis_name="s",
        num_cores=_NC, num_subcores=_NS)
    sc = pl.kernel(
        _sc_body,
        out_type=jax.ShapeDtypeStruct((rows, D), tokens.dtype),
        mesh=mesh,
        scratch_types=[
            pltpu.VMEM((8,), jnp.int32),
            pltpu.VMEM((8, D), jnp.float32),
            pltpu.VMEM((_CHUNK, D), jnp.float32),
            pltpu.SemaphoreType.DMA,
        ],
    )
    out = sc(tid_vec, embed_weight, flat)
    return out.reshape(B, N, D)
